# baseline (speedup 1.0000x reference)
import functools

import jax
import jax.numpy as jnp
from jax import lax
from jax.experimental import pallas as pl
from jax.experimental.pallas import tpu as pltpu
from jax.experimental.pallas import tpu_sc as plsc

_LANES = 16


def _make_sc_add(B, S, D):
    NC, NS = 2, 16
    NW = NC * NS
    rows_per_w = S // NW
    CH = 32
    n_chunks = rows_per_w // CH
    chunk = CH * D
    half = chunk // 2
    n_k = n_chunks * B

    mesh = plsc.VectorSubcoreMesh(core_axis_name="c", subcore_axis_name="s")

    @functools.partial(
        pl.kernel,
        out_type=jax.ShapeDtypeStruct((B * S * D,), jnp.float32),
        mesh=mesh,
        scratch_types=[
            [pltpu.VMEM((chunk,), jnp.float32) for _ in range(2)],
            [pltpu.VMEM((half,), jnp.float32) for _ in range(2)],
            [pltpu.SemaphoreType.DMA for _ in range(2)],
            [pltpu.SemaphoreType.DMA for _ in range(2)],
            [pltpu.SemaphoreType.DMA for _ in range(2)],
        ],
    )
    def sc_add(x_hbm, pe_hbm, out_hbm, x_v, pe_v, sx, so, sp):
        wid = lax.axis_index("s") * NC + lax.axis_index("c")
        base0 = wid * rows_per_w * D

        def x_off(k):
            c = k // B
            b = k % B
            return b * S * D + base0 + c * chunk

        def pe_off(c, h):
            return base0 + c * chunk + h * half

        pltpu.async_copy(x_hbm.at[pl.ds(x_off(0), chunk)], x_v[0], sx[0])
        for h in range(2):
            pltpu.async_copy(
                pe_hbm.at[pl.ds(pe_off(0, h), half)], pe_v[h], sp[h]
            )

        def c_body(c, carry):
            for bb in range(B):
                k = c * B + bb
                j = bb % 2
                j2 = 1 - j
                if bb == 0:
                    for h in range(2):
                        pltpu.make_async_copy(
                            pe_hbm.at[pl.ds(pe_off(c, h), half)], pe_v[h],
                            sp[h],
                        ).wait()
                pltpu.make_async_copy(
                    x_hbm.at[pl.ds(x_off(k), chunk)], x_v[j], sx[j]
                ).wait()

                for h in range(2):
                    @plsc.parallel_loop(0, half // _LANES, unroll=8)
                    def _add(i):
                        sl = pl.ds(h * half + i * _LANES, _LANES)
                        pl_sl = pl.ds(i * _LANES, _LANES)
                        x_v[j][sl] = x_v[j][sl] + pe_v[h][pl_sl]

                pltpu.async_copy(
                    x_v[j], out_hbm.at[pl.ds(x_off(k), chunk)], so[j]
                )
                if bb == B - 1:
                    @pl.when(c + 1 < n_chunks)
                    def _():
                        for h in range(2):
                            pltpu.async_copy(
                                pe_hbm.at[pl.ds(pe_off(c + 1, h), half)],
                                pe_v[h], sp[h],
                            )
                @pl.when(k + 1 < n_k)
                def _():
                    @pl.when(k >= 1)
                    def _():
                        pltpu.make_async_copy(
                            x_v[j2], out_hbm.at[pl.ds(x_off(k - 1), chunk)],
                            so[j2],
                        ).wait()
                    pltpu.async_copy(
                        x_hbm.at[pl.ds(x_off(k + 1), chunk)], x_v[j2], sx[j2]
                    )
            return carry

        lax.fori_loop(0, n_chunks, c_body, 0)

        for k in (n_k - 2, n_k - 1):
            j = k % 2
            pltpu.make_async_copy(
                x_v[j], out_hbm.at[pl.ds(x_off(k), chunk)], so[j]
            ).wait()

    return sc_add


def kernel(x, pos_embedding):
    B, S, D = x.shape
    out = _make_sc_add(B, S, D)(x.reshape(-1), pos_embedding.reshape(-1))
    return out.reshape(B, S, D)
